# baseline (speedup 1.0000x reference)
import functools
import jax
import jax.numpy as jnp
from jax import lax
from jax.experimental import pallas as pl
from jax.experimental.pallas import tpu as pltpu

_N = 10000
_DEBUG_JNP_SCATTER = False
_E = 160000
_H = 128
_NB = 16


def _silu(x):
    return x * jax.nn.sigmoid(x)


def _proj_body(x_ref, ws_ref, bs_ref, wd_ref, bd_ref, ps_ref, pd_ref):
    x = x_ref[...]
    ps_ref[...] = jnp.dot(x, ws_ref[...], preferred_element_type=jnp.float32) + bs_ref[...]
    pd_ref[...] = jnp.dot(x, wd_ref[...], preferred_element_type=jnp.float32) + bd_ref[...]


def _proj(x, W_src, b_src, W_dst, b_dst):
    blk = 2000
    grid = _N // blk
    return pl.pallas_call(
        _proj_body,
        grid=(grid,),
        in_specs=[
            pl.BlockSpec((blk, _H), lambda i: (i, 0)),
            pl.BlockSpec((_H, _H), lambda i: (0, 0)),
            pl.BlockSpec((1, _H), lambda i: (0, 0)),
            pl.BlockSpec((_H, _H), lambda i: (0, 0)),
            pl.BlockSpec((1, _H), lambda i: (0, 0)),
        ],
        out_specs=[
            pl.BlockSpec((blk, _H), lambda i: (i, 0)),
            pl.BlockSpec((blk, _H), lambda i: (i, 0)),
        ],
        out_shape=[
            jax.ShapeDtypeStruct((_N, _H), jnp.float32),
            jax.ShapeDtypeStruct((_N, _H), jnp.float32),
        ],
    )(x, W_src.T, b_src[None, :], W_dst.T, b_dst[None, :])


def _edge_mlp_body(gs_ref, gd_ref, rbf_ref, b1_ref, b2_ref,
                   a1_ref, a2_ref, a3_ref, bf1_ref,
                   w2_ref, bf2_ref, msg_ref, w9_ref):
    t = jnp.dot(gs_ref[...], a1_ref[...], preferred_element_type=jnp.float32)
    t += jnp.dot(gd_ref[...], a2_ref[...], preferred_element_type=jnp.float32)
    t += jnp.dot(rbf_ref[...], a3_ref[...], preferred_element_type=jnp.float32)
    t += bf1_ref[...]
    t = _silu(t)
    msg_ref[...] = (
        jnp.dot(t, w2_ref[...], preferred_element_type=jnp.float32) + bf2_ref[...]
    )
    blk = b1_ref.shape[0]
    w9_ref[...] = jnp.concatenate(
        [jnp.full((blk, 1), 1.0 / 16.0, jnp.float32), b1_ref[...], b2_ref[...],
         jnp.zeros((blk, _H - 9), jnp.float32)], axis=-1)


def _edge_mlp(gs, gd, rbf, basis_1, basis_2, Wf1, bf1, Wf2, bf2):
    A1 = Wf1[:, 0:_H].T
    A2 = Wf1[:, _H:2 * _H].T
    A3 = Wf1[:, 2 * _H:].T
    blk = 2000
    grid = _E // blk
    return pl.pallas_call(
        _edge_mlp_body,
        grid=(grid,),
        in_specs=[
            pl.BlockSpec((blk, _H), lambda i: (i, 0)),
            pl.BlockSpec((blk, _H), lambda i: (i, 0)),
            pl.BlockSpec((blk, _NB), lambda i: (i, 0)),
            pl.BlockSpec((blk, 3), lambda i: (i, 0)),
            pl.BlockSpec((blk, 5), lambda i: (i, 0)),
            pl.BlockSpec((_H, _H), lambda i: (0, 0)),
            pl.BlockSpec((_H, _H), lambda i: (0, 0)),
            pl.BlockSpec((_NB, _H), lambda i: (0, 0)),
            pl.BlockSpec((1, _H), lambda i: (0, 0)),
            pl.BlockSpec((_H, _H), lambda i: (0, 0)),
            pl.BlockSpec((1, _H), lambda i: (0, 0)),
        ],
        out_specs=[
            pl.BlockSpec((blk, _H), lambda i: (i, 0)),
            pl.BlockSpec((blk, _H), lambda i: (i, 0)),
        ],
        out_shape=[
            jax.ShapeDtypeStruct((_E, _H), jnp.float32),
            jax.ShapeDtypeStruct((_E, _H), jnp.float32),
        ],
    )(gs, gd, rbf, basis_1, basis_2, A1, A2, A3, bf1[None, :], Wf2.T,
      bf2[None, :])


def _update_body(x_ref, aggs_ref, u1_ref, bu1_ref, u2_ref, bu2_ref,
                 g_ref, b_ref, out_ref):
    x = x_ref[...]
    s = aggs_ref[:, 0:_H]
    i1 = aggs_ref[:, _H:2 * _H]
    i2 = aggs_ref[:, 2 * _H:3 * _H]
    u1 = u1_ref[...]
    t = jnp.dot(x, u1[0:_H, :], preferred_element_type=jnp.float32)
    t += jnp.dot(s, u1[_H:2 * _H, :], preferred_element_type=jnp.float32)
    t += jnp.dot(i1, u1[2 * _H:3 * _H, :], preferred_element_type=jnp.float32)
    t += jnp.dot(i2, u1[3 * _H:4 * _H, :], preferred_element_type=jnp.float32)
    t += bu1_ref[...]
    t = _silu(t)
    upd = jnp.dot(t, u2_ref[...], preferred_element_type=jnp.float32) + bu2_ref[...]
    y = x + upd
    mu = jnp.mean(y, axis=-1, keepdims=True)
    var = jnp.mean((y - mu) ** 2, axis=-1, keepdims=True)
    out_ref[...] = (y - mu) * lax.rsqrt(var + 1e-5) * g_ref[...] + b_ref[...]


def _update(x, aggs, Wu1, bu1, Wu2, bu2, ln_gamma, ln_beta):
    blk = 2000
    grid = _N // blk
    return pl.pallas_call(
        _update_body,
        grid=(grid,),
        in_specs=[
            pl.BlockSpec((blk, _H), lambda i: (i, 0)),
            pl.BlockSpec((blk, 3 * _H), lambda i: (i, 0)),
            pl.BlockSpec((4 * _H, 2 * _H), lambda i: (0, 0)),
            pl.BlockSpec((1, 2 * _H), lambda i: (0, 0)),
            pl.BlockSpec((2 * _H, _H), lambda i: (0, 0)),
            pl.BlockSpec((1, _H), lambda i: (0, 0)),
            pl.BlockSpec((1, _H), lambda i: (0, 0)),
            pl.BlockSpec((1, _H), lambda i: (0, 0)),
        ],
        out_specs=pl.BlockSpec((blk, _H), lambda i: (i, 0)),
        out_shape=jax.ShapeDtypeStruct((_N, _H), jnp.float32),
    )(x, aggs, Wu1.T, bu1[None, :], Wu2.T, bu2[None, :],
      ln_gamma[None, :], ln_beta[None, :])


from jax.experimental.pallas import tpu_sc as plsc

_NC = 2
_NS = 16
_NW = _NC * _NS
_EPW = _E // _NW
_GB = 200

_ES = _E // _NS
_NCHUNK = 40
_CHUNK = 256
_AROWS = 264
_DUMP = 256
_RPT = 16
_K = 16
_HALVES = (2496, 2496, 2496, 2512)
_CAP = 2512 // _K + 3


def _sc_gather(P_src, P_dst, src, dst):
    mesh = plsc.VectorSubcoreMesh(core_axis_name="c", subcore_axis_name="s")

    @functools.partial(
        pl.kernel,
        compiler_params=pltpu.CompilerParams(needs_layout_passes=False),
        out_type=[
            jax.ShapeDtypeStruct((_E, _H), jnp.float32),
            jax.ShapeDtypeStruct((_E, _H), jnp.float32),
        ],
        mesh=mesh,
        scratch_types=[
            pltpu.VMEM((_GB,), jnp.int32),
            pltpu.VMEM((_GB,), jnp.int32),
            pltpu.VMEM((_GB, _H), jnp.float32),
            pltpu.VMEM((_GB, _H), jnp.float32),
            pltpu.SemaphoreType.DMA,
            pltpu.SemaphoreType.DMA,
        ],
    )
    def gather_k(ps_hbm, pd_hbm, src_hbm, dst_hbm, gs_hbm, gd_hbm,
                 si_v, di_v, rs_v, rd_v, sem1, sem2):
        wid = lax.axis_index("s") * _NC + lax.axis_index("c")
        base = wid * _EPW

        def body(b, carry):
            off = base + b * _GB
            pltpu.sync_copy(src_hbm.at[pl.ds(off, _GB)], si_v)
            pltpu.sync_copy(dst_hbm.at[pl.ds(off, _GB)], di_v)
            c1 = pltpu.async_copy(ps_hbm.at[si_v], rs_v, sem1)
            c2 = pltpu.async_copy(pd_hbm.at[di_v], rd_v, sem2)
            c1.wait()
            c2.wait()
            pltpu.sync_copy(rs_v, gs_hbm.at[pl.ds(off, _GB)])
            pltpu.sync_copy(rd_v, gd_hbm.at[pl.ds(off, _GB)])
            return carry

        lax.fori_loop(0, _EPW // _GB, body, 0)

    return gather_k(P_src, P_dst, src, dst)


def _sc_scatter(msg, w9, dst):
    mesh = plsc.VectorSubcoreMesh(core_axis_name="c", subcore_axis_name="s")

    @functools.partial(
        pl.kernel,
        compiler_params=pltpu.CompilerParams(needs_layout_passes=False),
        out_type=jax.ShapeDtypeStruct((_N, 3 * _H), jnp.float32),
        mesh=mesh,
        scratch_types=[
            pltpu.VMEM((2512,), jnp.int32),
            pltpu.VMEM((_CAP, _K), jnp.int32),
            pltpu.VMEM((_CAP, _K), jnp.int32),
            pltpu.VMEM((_K, _H), jnp.float32),
            pltpu.VMEM((_K, _H), jnp.float32),
            pltpu.VMEM((_K, _H), jnp.float32),
            pltpu.VMEM((_K, _H), jnp.float32),
            pltpu.VMEM((9, _K, _H), jnp.float32),
            pltpu.VMEM((8, 3 * _H), jnp.float32),
        ] + [pltpu.VMEM_SHARED((_AROWS, _H), jnp.float32)] * 9 + [
            pltpu.SemaphoreType.DMA,
            pltpu.SemaphoreType.DMA,
            pltpu.SemaphoreType.DMA,
        ],
    )
    def scatter_k(msg_hbm, w9_hbm, dst_hbm, out_hbm,
                  dst_v, eid_v, dl_v, msg_v, w_v, msg2_v, w2_v, srow_v,
                  out_v,
                  a0, a1, a2, a3, a4, a5, a6, a7, a8, sem1, sem2, sem3):
        accs = [a0, a1, a2, a3, a4, a5, a6, a7, a8]
        cid = lax.axis_index("c")
        sid = lax.axis_index("s")
        sbase = sid * _ES
        iota16 = lax.broadcasted_iota(jnp.int32, (16,), 0)

        def chunk_body(j, carry):
            kk = cid * (_NCHUNK // _NC) + j
            lo = kk * _CHUNK
            hi = jnp.minimum(_N, lo + _CHUNK)
            sz = hi - lo

            for r in range(_K):
                for q in range(_H // 16):
                    srow_v[0, r, pl.ds(q * 16, 16)] = jnp.zeros(
                        (16,), jnp.float32)
            zbase = sid * _RPT
            for c in range(9):
                for i in range(_RPT // 8):
                    pltpu.sync_copy(srow_v.at[0, pl.ds(0, 8)],
                                    accs[c].at[pl.ds(zbase + 8 * i, 8)])
            plsc.subcore_barrier()

            hoff = 0
            for hlen in _HALVES:
                hbase = sbase + hoff
                pltpu.sync_copy(dst_hbm.at[pl.ds(hbase, hlen)],
                                dst_v.at[pl.ds(0, hlen)])

                def compact_body(i, cnt):
                    d16 = dst_v[pl.ds(i * 16, 16)]
                    m = (d16 >= lo) & (d16 < hi)
                    mi = jnp.where(m, 1, 0)
                    raw = cnt + plsc.cumsum(mi) - 1
                    idx = jnp.where(m, raw, (_CAP - 1) * 16 + iota16)
                    eids = iota16 + (hbase + i * 16)
                    plsc.store_scatter(eid_v, [idx >> 4, idx & 15], eids,
                                       mask=m)
                    plsc.store_scatter(dl_v, [idx >> 4, idx & 15],
                                       jnp.where(m, d16 - lo, _DUMP), mask=m)
                    return cnt + jnp.sum(mi)

                cnt = lax.fori_loop(0, hlen // 16, compact_body, 0)
                idxp = cnt + iota16
                plsc.store_scatter(eid_v, [idxp >> 4, idxp & 15],
                                   jnp.zeros((16,), jnp.int32))
                plsc.store_scatter(dl_v, [idxp >> 4, idxp & 15],
                                   jnp.full((16,), _DUMP, jnp.int32))

                def batch_body(b, carry):
                    c1 = pltpu.async_copy(msg_hbm.at[eid_v.at[b]], msg_v,
                                          sem1)
                    c2 = pltpu.async_copy(w9_hbm.at[eid_v.at[b]], w_v, sem2)
                    c1.wait()
                    c2.wait()

                    def row_body(r, carry2):
                        wvec = w_v[r, pl.ds(0, 16)]
                        for c in range(9):
                            wc = wvec[c]
                            for q in range(_H // 16):
                                srow_v[c, r, pl.ds(q * 16, 16)] = (
                                    wc * msg_v[r, pl.ds(q * 16, 16)])
                        return carry2

                    lax.fori_loop(0, _K, row_body, 0)
                    descs = []
                    for c in range(9):
                        d = pltpu.make_async_copy(
                            srow_v.at[c], accs[c].at[dl_v.at[b]], sem3)
                        d.start(add=True)
                        descs.append(d)
                    for d in descs:
                        d.wait()
                    return carry

                nb = (cnt + _K - 1) // _K
                lax.fori_loop(0, nb, batch_body, 0)
                hoff += hlen
            plsc.subcore_barrier()

            rstart = sid * _RPT
            ng = jnp.clip((sz - rstart) // 8, 0, _RPT // 8)

            def red_body(g, carry):
                row = rstart + 8 * g
                descs = [
                    pltpu.async_copy(accs[c].at[pl.ds(row, 8)],
                                     srow_v.at[c, pl.ds(0, 8)], sem3)
                    for c in range(9)
                ]
                for d in descs:
                    d.wait()

                def rrow(r, carry2):
                    for q in range(_H // 16):
                        s0 = srow_v[0, r, pl.ds(q * 16, 16)]
                        t1 = srow_v[1, r, pl.ds(q * 16, 16)]
                        t2 = srow_v[2, r, pl.ds(q * 16, 16)]
                        t3 = srow_v[3, r, pl.ds(q * 16, 16)]
                        i1 = (t1 * t1 + t2 * t2 + t3 * t3) * (1.0 / 3.0)
                        u1 = srow_v[4, r, pl.ds(q * 16, 16)]
                        u2 = srow_v[5, r, pl.ds(q * 16, 16)]
                        u3 = srow_v[6, r, pl.ds(q * 16, 16)]
                        u4 = srow_v[7, r, pl.ds(q * 16, 16)]
                        u5 = srow_v[8, r, pl.ds(q * 16, 16)]
                        i2 = (u1 * u1 + u2 * u2 + u3 * u3 + u4 * u4
                              + u5 * u5) * (1.0 / 5.0)
                        out_v[r, pl.ds(q * 16, 16)] = s0
                        out_v[r, pl.ds(_H + q * 16, 16)] = i1
                        out_v[r, pl.ds(2 * _H + q * 16, 16)] = i2
                    return carry2

                lax.fori_loop(0, 8, rrow, 0)
                pltpu.sync_copy(out_v, out_hbm.at[pl.ds(lo + row, 8)])
                return carry

            lax.fori_loop(0, ng, red_body, 0)
            plsc.subcore_barrier()
            return carry

        lax.fori_loop(0, _NCHUNK // _NC, chunk_body, 0)

    return scatter_k(msg, w9, dst)


def kernel(node_features, edge_index, edge_rbf, basis_0, basis_1, basis_2,
           W_src, b_src, W_dst, b_dst, Wf1, bf1, Wf2, bf2,
           Wu1, bu1, Wu2, bu2, ln_gamma, ln_beta):
    src = edge_index[0]
    dst = edge_index[1]
    P_src, P_dst = _proj(node_features, W_src, b_src, W_dst, b_dst)
    gs, gd = _sc_gather(P_src, P_dst, src, dst)
    msg, w9 = _edge_mlp(gs, gd, edge_rbf, basis_1, basis_2, Wf1, bf1, Wf2, bf2)
    if _DEBUG_JNP_SCATTER:
        prod = msg[:, None, :] * w9[:, 0:9, None]
        sums = jax.ops.segment_sum(prod, dst, num_segments=_N)
        jaggs = jnp.concatenate(
            [sums[:, 0, :], jnp.mean(sums[:, 1:4, :] ** 2, axis=1),
             jnp.mean(sums[:, 4:9, :] ** 2, axis=1)], axis=-1)
        saggs = _sc_scatter(msg, w9, dst)
        aggs = jnp.concatenate(
            [saggs[:, 0:128], jaggs[:, 128:384]], axis=-1)
    else:
        aggs = _sc_scatter(msg, w9, dst)
    return _update(node_features, aggs, Wu1, bu1, Wu2, bu2, ln_gamma, ln_beta)


# speedup vs baseline: 1.0047x; 1.0047x over previous
import functools
import jax
import jax.numpy as jnp
from jax import lax
from jax.experimental import pallas as pl
from jax.experimental.pallas import tpu as pltpu

_N = 10000
_DEBUG_JNP_SCATTER = False
_E = 160000
_H = 128
_NB = 16


def _silu(x):
    return x * jax.nn.sigmoid(x)


def _proj_body(x_ref, ws_ref, bs_ref, wd_ref, bd_ref, ps_ref, pd_ref):
    x = x_ref[...]
    ps_ref[...] = jnp.dot(x, ws_ref[...], preferred_element_type=jnp.float32) + bs_ref[...]
    pd_ref[...] = jnp.dot(x, wd_ref[...], preferred_element_type=jnp.float32) + bd_ref[...]


def _proj(x, W_src, b_src, W_dst, b_dst):
    blk = 2000
    grid = _N // blk
    return pl.pallas_call(
        _proj_body,
        grid=(grid,),
        in_specs=[
            pl.BlockSpec((blk, _H), lambda i: (i, 0)),
            pl.BlockSpec((_H, _H), lambda i: (0, 0)),
            pl.BlockSpec((1, _H), lambda i: (0, 0)),
            pl.BlockSpec((_H, _H), lambda i: (0, 0)),
            pl.BlockSpec((1, _H), lambda i: (0, 0)),
        ],
        out_specs=[
            pl.BlockSpec((blk, _H), lambda i: (i, 0)),
            pl.BlockSpec((blk, _H), lambda i: (i, 0)),
        ],
        out_shape=[
            jax.ShapeDtypeStruct((_N, _H), jnp.float32),
            jax.ShapeDtypeStruct((_N, _H), jnp.float32),
        ],
    )(x, W_src.T, b_src[None, :], W_dst.T, b_dst[None, :])


def _edge_mlp_body(gs_ref, gd_ref, rbf_ref, b1_ref, b2_ref,
                   a1_ref, a2_ref, a3_ref, bf1_ref,
                   w2_ref, bf2_ref, msg_ref, w9_ref):
    t = jnp.dot(gs_ref[...], a1_ref[...], preferred_element_type=jnp.float32)
    t += jnp.dot(gd_ref[...], a2_ref[...], preferred_element_type=jnp.float32)
    t += jnp.dot(rbf_ref[...], a3_ref[...], preferred_element_type=jnp.float32)
    t += bf1_ref[...]
    t = _silu(t)
    msg_ref[...] = (
        jnp.dot(t, w2_ref[...], preferred_element_type=jnp.float32) + bf2_ref[...]
    )
    blk = b1_ref.shape[0]
    w9_ref[...] = jnp.concatenate(
        [jnp.full((blk, 1), 1.0 / 16.0, jnp.float32), b1_ref[...], b2_ref[...],
         jnp.zeros((blk, _H - 9), jnp.float32)], axis=-1)


def _edge_mlp(gs, gd, rbf, basis_1, basis_2, Wf1, bf1, Wf2, bf2):
    A1 = Wf1[:, 0:_H].T
    A2 = Wf1[:, _H:2 * _H].T
    A3 = Wf1[:, 2 * _H:].T
    blk = 2000
    grid = _E // blk
    return pl.pallas_call(
        _edge_mlp_body,
        grid=(grid,),
        in_specs=[
            pl.BlockSpec((blk, _H), lambda i: (i, 0)),
            pl.BlockSpec((blk, _H), lambda i: (i, 0)),
            pl.BlockSpec((blk, _NB), lambda i: (i, 0)),
            pl.BlockSpec((blk, 3), lambda i: (i, 0)),
            pl.BlockSpec((blk, 5), lambda i: (i, 0)),
            pl.BlockSpec((_H, _H), lambda i: (0, 0)),
            pl.BlockSpec((_H, _H), lambda i: (0, 0)),
            pl.BlockSpec((_NB, _H), lambda i: (0, 0)),
            pl.BlockSpec((1, _H), lambda i: (0, 0)),
            pl.BlockSpec((_H, _H), lambda i: (0, 0)),
            pl.BlockSpec((1, _H), lambda i: (0, 0)),
        ],
        out_specs=[
            pl.BlockSpec((blk, _H), lambda i: (i, 0)),
            pl.BlockSpec((blk, _H), lambda i: (i, 0)),
        ],
        out_shape=[
            jax.ShapeDtypeStruct((_E, _H), jnp.float32),
            jax.ShapeDtypeStruct((_E, _H), jnp.float32),
        ],
    )(gs, gd, rbf, basis_1, basis_2, A1, A2, A3, bf1[None, :], Wf2.T,
      bf2[None, :])


def _update_body(x_ref, aggs_ref, u1_ref, bu1_ref, u2_ref, bu2_ref,
                 g_ref, b_ref, out_ref):
    x = x_ref[...]
    s = aggs_ref[:, 0:_H]
    i1 = aggs_ref[:, _H:2 * _H]
    i2 = aggs_ref[:, 2 * _H:3 * _H]
    u1 = u1_ref[...]
    t = jnp.dot(x, u1[0:_H, :], preferred_element_type=jnp.float32)
    t += jnp.dot(s, u1[_H:2 * _H, :], preferred_element_type=jnp.float32)
    t += jnp.dot(i1, u1[2 * _H:3 * _H, :], preferred_element_type=jnp.float32)
    t += jnp.dot(i2, u1[3 * _H:4 * _H, :], preferred_element_type=jnp.float32)
    t += bu1_ref[...]
    t = _silu(t)
    upd = jnp.dot(t, u2_ref[...], preferred_element_type=jnp.float32) + bu2_ref[...]
    y = x + upd
    mu = jnp.mean(y, axis=-1, keepdims=True)
    var = jnp.mean((y - mu) ** 2, axis=-1, keepdims=True)
    out_ref[...] = (y - mu) * lax.rsqrt(var + 1e-5) * g_ref[...] + b_ref[...]


def _update(x, aggs, Wu1, bu1, Wu2, bu2, ln_gamma, ln_beta):
    blk = 2000
    grid = _N // blk
    return pl.pallas_call(
        _update_body,
        grid=(grid,),
        in_specs=[
            pl.BlockSpec((blk, _H), lambda i: (i, 0)),
            pl.BlockSpec((blk, 3 * _H), lambda i: (i, 0)),
            pl.BlockSpec((4 * _H, 2 * _H), lambda i: (0, 0)),
            pl.BlockSpec((1, 2 * _H), lambda i: (0, 0)),
            pl.BlockSpec((2 * _H, _H), lambda i: (0, 0)),
            pl.BlockSpec((1, _H), lambda i: (0, 0)),
            pl.BlockSpec((1, _H), lambda i: (0, 0)),
            pl.BlockSpec((1, _H), lambda i: (0, 0)),
        ],
        out_specs=pl.BlockSpec((blk, _H), lambda i: (i, 0)),
        out_shape=jax.ShapeDtypeStruct((_N, _H), jnp.float32),
    )(x, aggs, Wu1.T, bu1[None, :], Wu2.T, bu2[None, :],
      ln_gamma[None, :], ln_beta[None, :])


from jax.experimental.pallas import tpu_sc as plsc

_NC = 2
_NS = 16
_NW = _NC * _NS
_EPW = _E // _NW
_GB = 200

_ES = _E // _NS
_NCHUNK = 40
_CHUNK = 256
_AROWS = 264
_DUMP = 256
_RPT = 16
_K = 16
_HALVES = (2496, 2496, 2496, 2512)
_CAP = 2512 // _K + 3


def _sc_gather(P_src, P_dst, src, dst):
    mesh = plsc.VectorSubcoreMesh(core_axis_name="c", subcore_axis_name="s")

    @functools.partial(
        pl.kernel,
        compiler_params=pltpu.CompilerParams(needs_layout_passes=False),
        out_type=[
            jax.ShapeDtypeStruct((_E, _H), jnp.float32),
            jax.ShapeDtypeStruct((_E, _H), jnp.float32),
        ],
        mesh=mesh,
        scratch_types=[
            pltpu.VMEM((_GB,), jnp.int32),
            pltpu.VMEM((_GB,), jnp.int32),
            pltpu.VMEM((_GB, _H), jnp.float32),
            pltpu.VMEM((_GB, _H), jnp.float32),
            pltpu.SemaphoreType.DMA,
            pltpu.SemaphoreType.DMA,
        ],
    )
    def gather_k(ps_hbm, pd_hbm, src_hbm, dst_hbm, gs_hbm, gd_hbm,
                 si_v, di_v, rs_v, rd_v, sem1, sem2):
        wid = lax.axis_index("s") * _NC + lax.axis_index("c")
        base = wid * _EPW

        def body(b, carry):
            off = base + b * _GB
            pltpu.sync_copy(src_hbm.at[pl.ds(off, _GB)], si_v)
            pltpu.sync_copy(dst_hbm.at[pl.ds(off, _GB)], di_v)
            c1 = pltpu.async_copy(ps_hbm.at[si_v], rs_v, sem1)
            c2 = pltpu.async_copy(pd_hbm.at[di_v], rd_v, sem2)
            c1.wait()
            c2.wait()
            pltpu.sync_copy(rs_v, gs_hbm.at[pl.ds(off, _GB)])
            pltpu.sync_copy(rd_v, gd_hbm.at[pl.ds(off, _GB)])
            return carry

        lax.fori_loop(0, _EPW // _GB, body, 0)

    return gather_k(P_src, P_dst, src, dst)


def _sc_scatter(msg, w9, dst):
    mesh = plsc.VectorSubcoreMesh(core_axis_name="c", subcore_axis_name="s")

    @functools.partial(
        pl.kernel,
        compiler_params=pltpu.CompilerParams(needs_layout_passes=False),
        out_type=jax.ShapeDtypeStruct((_N, 3 * _H), jnp.float32),
        mesh=mesh,
        scratch_types=[
            pltpu.VMEM((2512,), jnp.int32),
            pltpu.VMEM((_CAP, _K), jnp.int32),
            pltpu.VMEM((_CAP, _K), jnp.int32),
            pltpu.VMEM((_K, _H), jnp.float32),
            pltpu.VMEM((_K, _H), jnp.float32),
            pltpu.VMEM((_K, _H), jnp.float32),
            pltpu.VMEM((_K, _H), jnp.float32),
            pltpu.VMEM((9, _K, _H), jnp.float32),
            pltpu.VMEM((9, _K, _H), jnp.float32),
            pltpu.VMEM((8, 3 * _H), jnp.float32),
        ] + [pltpu.VMEM_SHARED((_AROWS, _H), jnp.float32)] * 9 + [
            pltpu.SemaphoreType.DMA,
            pltpu.SemaphoreType.DMA,
            pltpu.SemaphoreType.DMA,
            pltpu.SemaphoreType.DMA,
        ],
    )
    def scatter_k(msg_hbm, w9_hbm, dst_hbm, out_hbm,
                  dst_v, eid_v, dl_v, msg_v, w_v, msg2_v, w2_v, srow_v,
                  srow2_v, out_v,
                  a0, a1, a2, a3, a4, a5, a6, a7, a8, sem1, sem2, sem3,
                  sem4):
        accs = [a0, a1, a2, a3, a4, a5, a6, a7, a8]
        cid = lax.axis_index("c")
        sid = lax.axis_index("s")
        sbase = sid * _ES
        iota16 = lax.broadcasted_iota(jnp.int32, (16,), 0)

        def chunk_body(j, carry):
            kk = cid * (_NCHUNK // _NC) + j
            lo = kk * _CHUNK
            hi = jnp.minimum(_N, lo + _CHUNK)
            sz = hi - lo

            for r in range(_K):
                for q in range(_H // 16):
                    srow_v[0, r, pl.ds(q * 16, 16)] = jnp.zeros(
                        (16,), jnp.float32)
            zbase = sid * _RPT
            for c in range(9):
                for i in range(_RPT // 8):
                    pltpu.sync_copy(srow_v.at[0, pl.ds(0, 8)],
                                    accs[c].at[pl.ds(zbase + 8 * i, 8)])
            plsc.subcore_barrier()

            hoff = 0
            for hlen in _HALVES:
                hbase = sbase + hoff
                pltpu.sync_copy(dst_hbm.at[pl.ds(hbase, hlen)],
                                dst_v.at[pl.ds(0, hlen)])

                def compact_body(i, cnt):
                    d16 = dst_v[pl.ds(i * 16, 16)]
                    m = (d16 >= lo) & (d16 < hi)
                    mi = jnp.where(m, 1, 0)
                    raw = cnt + plsc.cumsum(mi) - 1
                    idx = jnp.where(m, raw, (_CAP - 1) * 16 + iota16)
                    eids = iota16 + (hbase + i * 16)
                    plsc.store_scatter(eid_v, [idx >> 4, idx & 15], eids,
                                       mask=m)
                    plsc.store_scatter(dl_v, [idx >> 4, idx & 15],
                                       jnp.where(m, d16 - lo, _DUMP), mask=m)
                    return cnt + jnp.sum(mi)

                cnt = lax.fori_loop(0, hlen // 16, compact_body, 0)
                idxp = cnt + iota16
                plsc.store_scatter(eid_v, [idxp >> 4, idxp & 15],
                                   jnp.zeros((16,), jnp.int32))
                plsc.store_scatter(dl_v, [idxp >> 4, idxp & 15],
                                   jnp.full((16,), _DUMP, jnp.int32))

                def batch_body(b, carry):
                    c1 = pltpu.async_copy(msg_hbm.at[eid_v.at[b]], msg_v,
                                          sem1)
                    c2 = pltpu.async_copy(w9_hbm.at[eid_v.at[b]], w_v, sem2)
                    c1.wait()
                    c2.wait()

                    def row_body(r, carry2):
                        wvec = w_v[r, pl.ds(0, 16)]
                        for c in range(9):
                            wc = wvec[c]
                            for q in range(_H // 16):
                                srow_v[c, r, pl.ds(q * 16, 16)] = (
                                    wc * msg_v[r, pl.ds(q * 16, 16)])
                        return carry2

                    lax.fori_loop(0, _K, row_body, 0)
                    descs = []
                    for c in range(9):
                        d = pltpu.make_async_copy(
                            srow_v.at[c], accs[c].at[dl_v.at[b]], sem3)
                        d.start(add=True)
                        descs.append(d)
                    for d in descs:
                        d.wait()
                    return carry

                nb = (cnt + _K - 1) // _K
                lax.fori_loop(0, nb, batch_body, 0)
                hoff += hlen
            plsc.subcore_barrier()

            rstart = sid * _RPT
            ng = jnp.clip((sz - rstart) // 8, 0, _RPT // 8)

            def red_body(g, carry):
                row = rstart + 8 * g
                descs = [
                    pltpu.async_copy(accs[c].at[pl.ds(row, 8)],
                                     srow_v.at[c, pl.ds(0, 8)], sem3)
                    for c in range(9)
                ]
                for d in descs:
                    d.wait()

                def rrow(r, carry2):
                    for q in range(_H // 16):
                        s0 = srow_v[0, r, pl.ds(q * 16, 16)]
                        t1 = srow_v[1, r, pl.ds(q * 16, 16)]
                        t2 = srow_v[2, r, pl.ds(q * 16, 16)]
                        t3 = srow_v[3, r, pl.ds(q * 16, 16)]
                        i1 = (t1 * t1 + t2 * t2 + t3 * t3) * (1.0 / 3.0)
                        u1 = srow_v[4, r, pl.ds(q * 16, 16)]
                        u2 = srow_v[5, r, pl.ds(q * 16, 16)]
                        u3 = srow_v[6, r, pl.ds(q * 16, 16)]
                        u4 = srow_v[7, r, pl.ds(q * 16, 16)]
                        u5 = srow_v[8, r, pl.ds(q * 16, 16)]
                        i2 = (u1 * u1 + u2 * u2 + u3 * u3 + u4 * u4
                              + u5 * u5) * (1.0 / 5.0)
                        out_v[r, pl.ds(q * 16, 16)] = s0
                        out_v[r, pl.ds(_H + q * 16, 16)] = i1
                        out_v[r, pl.ds(2 * _H + q * 16, 16)] = i2
                    return carry2

                lax.fori_loop(0, 8, rrow, 0)
                pltpu.sync_copy(out_v, out_hbm.at[pl.ds(lo + row, 8)])
                return carry

            lax.fori_loop(0, ng, red_body, 0)
            plsc.subcore_barrier()
            return carry

        lax.fori_loop(0, _NCHUNK // _NC, chunk_body, 0)

    return scatter_k(msg, w9, dst)


def kernel(node_features, edge_index, edge_rbf, basis_0, basis_1, basis_2,
           W_src, b_src, W_dst, b_dst, Wf1, bf1, Wf2, bf2,
           Wu1, bu1, Wu2, bu2, ln_gamma, ln_beta):
    src = edge_index[0]
    dst = edge_index[1]
    P_src, P_dst = _proj(node_features, W_src, b_src, W_dst, b_dst)
    gs, gd = _sc_gather(P_src, P_dst, src, dst)
    msg, w9 = _edge_mlp(gs, gd, edge_rbf, basis_1, basis_2, Wf1, bf1, Wf2, bf2)
    if _DEBUG_JNP_SCATTER:
        prod = msg[:, None, :] * w9[:, 0:9, None]
        sums = jax.ops.segment_sum(prod, dst, num_segments=_N)
        jaggs = jnp.concatenate(
            [sums[:, 0, :], jnp.mean(sums[:, 1:4, :] ** 2, axis=1),
             jnp.mean(sums[:, 4:9, :] ** 2, axis=1)], axis=-1)
        saggs = _sc_scatter(msg, w9, dst)
        aggs = jnp.concatenate(
            [saggs[:, 0:128], jaggs[:, 128:384]], axis=-1)
    else:
        aggs = _sc_scatter(msg, w9, dst)
    return _update(node_features, aggs, Wu1, bu1, Wu2, bu2, ln_gamma, ln_beta)


# speedup vs baseline: 1.0921x; 1.0871x over previous
import functools
import jax
import jax.numpy as jnp
from jax import lax
from jax.experimental import pallas as pl
from jax.experimental.pallas import tpu as pltpu

_N = 10000
_DEBUG_JNP_SCATTER = False
_E = 160000
_H = 128
_NB = 16


def _silu(x):
    return x * jax.nn.sigmoid(x)


def _proj_body(x_ref, ws_ref, bs_ref, wd_ref, bd_ref, ps_ref, pd_ref):
    x = x_ref[...]
    ps_ref[...] = jnp.dot(x, ws_ref[...], preferred_element_type=jnp.float32) + bs_ref[...]
    pd_ref[...] = jnp.dot(x, wd_ref[...], preferred_element_type=jnp.float32) + bd_ref[...]


def _proj(x, W_src, b_src, W_dst, b_dst):
    blk = 2000
    grid = _N // blk
    return pl.pallas_call(
        _proj_body,
        grid=(grid,),
        in_specs=[
            pl.BlockSpec((blk, _H), lambda i: (i, 0)),
            pl.BlockSpec((_H, _H), lambda i: (0, 0)),
            pl.BlockSpec((1, _H), lambda i: (0, 0)),
            pl.BlockSpec((_H, _H), lambda i: (0, 0)),
            pl.BlockSpec((1, _H), lambda i: (0, 0)),
        ],
        out_specs=[
            pl.BlockSpec((blk, _H), lambda i: (i, 0)),
            pl.BlockSpec((blk, _H), lambda i: (i, 0)),
        ],
        out_shape=[
            jax.ShapeDtypeStruct((_N, _H), jnp.float32),
            jax.ShapeDtypeStruct((_N, _H), jnp.float32),
        ],
    )(x, W_src.T, b_src[None, :], W_dst.T, b_dst[None, :])


def _edge_mlp_body(gs_ref, gd_ref, rbf_ref, b1_ref, b2_ref,
                   a1_ref, a2_ref, a3_ref, bf1_ref,
                   w2_ref, bf2_ref, msg_ref, w9_ref):
    t = jnp.dot(gs_ref[...], a1_ref[...], preferred_element_type=jnp.float32)
    t += jnp.dot(gd_ref[...], a2_ref[...], preferred_element_type=jnp.float32)
    t += jnp.dot(rbf_ref[...], a3_ref[...], preferred_element_type=jnp.float32)
    t += bf1_ref[...]
    t = _silu(t)
    msg_ref[...] = (
        jnp.dot(t, w2_ref[...], preferred_element_type=jnp.float32) + bf2_ref[...]
    )
    blk = b1_ref.shape[0]
    w9_ref[...] = jnp.concatenate(
        [jnp.full((blk, 1), 1.0 / 16.0, jnp.float32), b1_ref[...], b2_ref[...],
         jnp.zeros((blk, _H - 9), jnp.float32)], axis=-1)


def _edge_mlp(gs, gd, rbf, basis_1, basis_2, Wf1, bf1, Wf2, bf2):
    A1 = Wf1[:, 0:_H].T
    A2 = Wf1[:, _H:2 * _H].T
    A3 = Wf1[:, 2 * _H:].T
    blk = 2000
    grid = _E // blk
    return pl.pallas_call(
        _edge_mlp_body,
        grid=(grid,),
        in_specs=[
            pl.BlockSpec((blk, _H), lambda i: (i, 0)),
            pl.BlockSpec((blk, _H), lambda i: (i, 0)),
            pl.BlockSpec((blk, _NB), lambda i: (i, 0)),
            pl.BlockSpec((blk, 3), lambda i: (i, 0)),
            pl.BlockSpec((blk, 5), lambda i: (i, 0)),
            pl.BlockSpec((_H, _H), lambda i: (0, 0)),
            pl.BlockSpec((_H, _H), lambda i: (0, 0)),
            pl.BlockSpec((_NB, _H), lambda i: (0, 0)),
            pl.BlockSpec((1, _H), lambda i: (0, 0)),
            pl.BlockSpec((_H, _H), lambda i: (0, 0)),
            pl.BlockSpec((1, _H), lambda i: (0, 0)),
        ],
        out_specs=[
            pl.BlockSpec((blk, _H), lambda i: (i, 0)),
            pl.BlockSpec((blk, _H), lambda i: (i, 0)),
        ],
        out_shape=[
            jax.ShapeDtypeStruct((_E, _H), jnp.float32),
            jax.ShapeDtypeStruct((_E, _H), jnp.float32),
        ],
    )(gs, gd, rbf, basis_1, basis_2, A1, A2, A3, bf1[None, :], Wf2.T,
      bf2[None, :])


def _update_body(x_ref, aggs_ref, u1_ref, bu1_ref, u2_ref, bu2_ref,
                 g_ref, b_ref, out_ref):
    x = x_ref[...]
    s = aggs_ref[:, 0:_H]
    i1 = aggs_ref[:, _H:2 * _H]
    i2 = aggs_ref[:, 2 * _H:3 * _H]
    u1 = u1_ref[...]
    t = jnp.dot(x, u1[0:_H, :], preferred_element_type=jnp.float32)
    t += jnp.dot(s, u1[_H:2 * _H, :], preferred_element_type=jnp.float32)
    t += jnp.dot(i1, u1[2 * _H:3 * _H, :], preferred_element_type=jnp.float32)
    t += jnp.dot(i2, u1[3 * _H:4 * _H, :], preferred_element_type=jnp.float32)
    t += bu1_ref[...]
    t = _silu(t)
    upd = jnp.dot(t, u2_ref[...], preferred_element_type=jnp.float32) + bu2_ref[...]
    y = x + upd
    mu = jnp.mean(y, axis=-1, keepdims=True)
    var = jnp.mean((y - mu) ** 2, axis=-1, keepdims=True)
    out_ref[...] = (y - mu) * lax.rsqrt(var + 1e-5) * g_ref[...] + b_ref[...]


def _update(x, aggs, Wu1, bu1, Wu2, bu2, ln_gamma, ln_beta):
    blk = 2000
    grid = _N // blk
    return pl.pallas_call(
        _update_body,
        grid=(grid,),
        in_specs=[
            pl.BlockSpec((blk, _H), lambda i: (i, 0)),
            pl.BlockSpec((blk, 3 * _H), lambda i: (i, 0)),
            pl.BlockSpec((4 * _H, 2 * _H), lambda i: (0, 0)),
            pl.BlockSpec((1, 2 * _H), lambda i: (0, 0)),
            pl.BlockSpec((2 * _H, _H), lambda i: (0, 0)),
            pl.BlockSpec((1, _H), lambda i: (0, 0)),
            pl.BlockSpec((1, _H), lambda i: (0, 0)),
            pl.BlockSpec((1, _H), lambda i: (0, 0)),
        ],
        out_specs=pl.BlockSpec((blk, _H), lambda i: (i, 0)),
        out_shape=jax.ShapeDtypeStruct((_N, _H), jnp.float32),
    )(x, aggs, Wu1.T, bu1[None, :], Wu2.T, bu2[None, :],
      ln_gamma[None, :], ln_beta[None, :])


from jax.experimental.pallas import tpu_sc as plsc

_NC = 2
_NS = 16
_NW = _NC * _NS
_EPW = _E // _NW
_GB = 200

_ES = _E // _NS
_NCHUNK = 28
_CHUNK = 384
_AROWS = 392
_DUMP = 384
_RPT = 24
_K = 16
_HALVES = (2496, 2496, 2496, 2512)
_CAP = 2512 // _K + 3


def _sc_gather(P_src, P_dst, src, dst):
    mesh = plsc.VectorSubcoreMesh(core_axis_name="c", subcore_axis_name="s")

    @functools.partial(
        pl.kernel,
        compiler_params=pltpu.CompilerParams(needs_layout_passes=False),
        out_type=[
            jax.ShapeDtypeStruct((_E, _H), jnp.float32),
            jax.ShapeDtypeStruct((_E, _H), jnp.float32),
        ],
        mesh=mesh,
        scratch_types=[
            pltpu.VMEM((_GB,), jnp.int32),
            pltpu.VMEM((_GB,), jnp.int32),
            pltpu.VMEM((_GB, _H), jnp.float32),
            pltpu.VMEM((_GB, _H), jnp.float32),
            pltpu.SemaphoreType.DMA,
            pltpu.SemaphoreType.DMA,
        ],
    )
    def gather_k(ps_hbm, pd_hbm, src_hbm, dst_hbm, gs_hbm, gd_hbm,
                 si_v, di_v, rs_v, rd_v, sem1, sem2):
        wid = lax.axis_index("s") * _NC + lax.axis_index("c")
        base = wid * _EPW

        def body(b, carry):
            off = base + b * _GB
            pltpu.sync_copy(src_hbm.at[pl.ds(off, _GB)], si_v)
            pltpu.sync_copy(dst_hbm.at[pl.ds(off, _GB)], di_v)
            c1 = pltpu.async_copy(ps_hbm.at[si_v], rs_v, sem1)
            c2 = pltpu.async_copy(pd_hbm.at[di_v], rd_v, sem2)
            c1.wait()
            c2.wait()
            pltpu.sync_copy(rs_v, gs_hbm.at[pl.ds(off, _GB)])
            pltpu.sync_copy(rd_v, gd_hbm.at[pl.ds(off, _GB)])
            return carry

        lax.fori_loop(0, _EPW // _GB, body, 0)

    return gather_k(P_src, P_dst, src, dst)


def _sc_scatter(msg, w9, dst):
    mesh = plsc.VectorSubcoreMesh(core_axis_name="c", subcore_axis_name="s")

    @functools.partial(
        pl.kernel,
        compiler_params=pltpu.CompilerParams(needs_layout_passes=False),
        out_type=jax.ShapeDtypeStruct((_N, 3 * _H), jnp.float32),
        mesh=mesh,
        scratch_types=[
            pltpu.VMEM((2512,), jnp.int32),
            pltpu.VMEM((_CAP, _K), jnp.int32),
            pltpu.VMEM((_CAP, _K), jnp.int32),
            pltpu.VMEM((_K, _H), jnp.float32),
            pltpu.VMEM((_K, _H), jnp.float32),
            pltpu.VMEM((_K, _H), jnp.float32),
            pltpu.VMEM((_K, _H), jnp.float32),
            pltpu.VMEM((9, _K, _H), jnp.float32),
            pltpu.VMEM((9, _K, _H), jnp.float32),
            pltpu.VMEM((8, 3 * _H), jnp.float32),
        ] + [pltpu.VMEM_SHARED((_AROWS, _H), jnp.float32)] * 9 + [
            pltpu.SemaphoreType.DMA,
            pltpu.SemaphoreType.DMA,
            pltpu.SemaphoreType.DMA,
            pltpu.SemaphoreType.DMA,
        ],
    )
    def scatter_k(msg_hbm, w9_hbm, dst_hbm, out_hbm,
                  dst_v, eid_v, dl_v, msg_v, w_v, msg2_v, w2_v, srow_v,
                  srow2_v, out_v,
                  a0, a1, a2, a3, a4, a5, a6, a7, a8, sem1, sem2, sem3,
                  sem4):
        accs = [a0, a1, a2, a3, a4, a5, a6, a7, a8]
        cid = lax.axis_index("c")
        sid = lax.axis_index("s")
        sbase = sid * _ES
        iota16 = lax.broadcasted_iota(jnp.int32, (16,), 0)

        def chunk_body(j, carry):
            kk = cid * (_NCHUNK // _NC) + j
            lo = kk * _CHUNK
            hi = jnp.minimum(_N, lo + _CHUNK)
            sz = hi - lo

            for r in range(_K):
                for q in range(_H // 16):
                    srow_v[0, r, pl.ds(q * 16, 16)] = jnp.zeros(
                        (16,), jnp.float32)
            zbase = sid * _RPT
            for c in range(9):
                for i in range(_RPT // 8):
                    pltpu.sync_copy(srow_v.at[0, pl.ds(0, 8)],
                                    accs[c].at[pl.ds(zbase + 8 * i, 8)])
            plsc.subcore_barrier()

            hoff = 0
            for hlen in _HALVES:
                hbase = sbase + hoff
                pltpu.sync_copy(dst_hbm.at[pl.ds(hbase, hlen)],
                                dst_v.at[pl.ds(0, hlen)])

                def compact_body(i, cnt):
                    d16 = dst_v[pl.ds(i * 16, 16)]
                    m = (d16 >= lo) & (d16 < hi)
                    mi = jnp.where(m, 1, 0)
                    raw = cnt + plsc.cumsum(mi) - 1
                    idx = jnp.where(m, raw, (_CAP - 1) * 16 + iota16)
                    eids = iota16 + (hbase + i * 16)
                    plsc.store_scatter(eid_v, [idx >> 4, idx & 15], eids,
                                       mask=m)
                    plsc.store_scatter(dl_v, [idx >> 4, idx & 15],
                                       jnp.where(m, d16 - lo, _DUMP), mask=m)
                    return cnt + jnp.sum(mi)

                cnt = lax.fori_loop(0, hlen // 16, compact_body, 0)
                idxp = cnt + iota16
                plsc.store_scatter(eid_v, [idxp >> 4, idxp & 15],
                                   jnp.zeros((16,), jnp.int32))
                plsc.store_scatter(dl_v, [idxp >> 4, idxp & 15],
                                   jnp.full((16,), _DUMP, jnp.int32))

                def batch_body(b, carry):
                    c1 = pltpu.async_copy(msg_hbm.at[eid_v.at[b]], msg_v,
                                          sem1)
                    c2 = pltpu.async_copy(w9_hbm.at[eid_v.at[b]], w_v, sem2)
                    c1.wait()
                    c2.wait()

                    def row_body(r, carry2):
                        wvec = w_v[r, pl.ds(0, 16)]
                        for c in range(9):
                            wc = wvec[c]
                            for q in range(_H // 16):
                                srow_v[c, r, pl.ds(q * 16, 16)] = (
                                    wc * msg_v[r, pl.ds(q * 16, 16)])
                        return carry2

                    lax.fori_loop(0, _K, row_body, 0)
                    descs = []
                    for c in range(9):
                        d = pltpu.make_async_copy(
                            srow_v.at[c], accs[c].at[dl_v.at[b]], sem3)
                        d.start(add=True)
                        descs.append(d)
                    for d in descs:
                        d.wait()
                    return carry

                nb = (cnt + _K - 1) // _K
                lax.fori_loop(0, nb, batch_body, 0)
                hoff += hlen
            plsc.subcore_barrier()

            rstart = sid * _RPT
            ng = jnp.clip((sz - rstart) // 8, 0, _RPT // 8)

            def red_body(g, carry):
                row = rstart + 8 * g
                descs = [
                    pltpu.async_copy(accs[c].at[pl.ds(row, 8)],
                                     srow_v.at[c, pl.ds(0, 8)], sem3)
                    for c in range(9)
                ]
                for d in descs:
                    d.wait()

                def rrow(r, carry2):
                    for q in range(_H // 16):
                        s0 = srow_v[0, r, pl.ds(q * 16, 16)]
                        t1 = srow_v[1, r, pl.ds(q * 16, 16)]
                        t2 = srow_v[2, r, pl.ds(q * 16, 16)]
                        t3 = srow_v[3, r, pl.ds(q * 16, 16)]
                        i1 = (t1 * t1 + t2 * t2 + t3 * t3) * (1.0 / 3.0)
                        u1 = srow_v[4, r, pl.ds(q * 16, 16)]
                        u2 = srow_v[5, r, pl.ds(q * 16, 16)]
                        u3 = srow_v[6, r, pl.ds(q * 16, 16)]
                        u4 = srow_v[7, r, pl.ds(q * 16, 16)]
                        u5 = srow_v[8, r, pl.ds(q * 16, 16)]
                        i2 = (u1 * u1 + u2 * u2 + u3 * u3 + u4 * u4
                              + u5 * u5) * (1.0 / 5.0)
                        out_v[r, pl.ds(q * 16, 16)] = s0
                        out_v[r, pl.ds(_H + q * 16, 16)] = i1
                        out_v[r, pl.ds(2 * _H + q * 16, 16)] = i2
                    return carry2

                lax.fori_loop(0, 8, rrow, 0)
                pltpu.sync_copy(out_v, out_hbm.at[pl.ds(lo + row, 8)])
                return carry

            lax.fori_loop(0, ng, red_body, 0)
            plsc.subcore_barrier()
            return carry

        lax.fori_loop(0, _NCHUNK // _NC, chunk_body, 0)

    return scatter_k(msg, w9, dst)


def kernel(node_features, edge_index, edge_rbf, basis_0, basis_1, basis_2,
           W_src, b_src, W_dst, b_dst, Wf1, bf1, Wf2, bf2,
           Wu1, bu1, Wu2, bu2, ln_gamma, ln_beta):
    src = edge_index[0]
    dst = edge_index[1]
    P_src, P_dst = _proj(node_features, W_src, b_src, W_dst, b_dst)
    gs, gd = _sc_gather(P_src, P_dst, src, dst)
    msg, w9 = _edge_mlp(gs, gd, edge_rbf, basis_1, basis_2, Wf1, bf1, Wf2, bf2)
    if _DEBUG_JNP_SCATTER:
        prod = msg[:, None, :] * w9[:, 0:9, None]
        sums = jax.ops.segment_sum(prod, dst, num_segments=_N)
        jaggs = jnp.concatenate(
            [sums[:, 0, :], jnp.mean(sums[:, 1:4, :] ** 2, axis=1),
             jnp.mean(sums[:, 4:9, :] ** 2, axis=1)], axis=-1)
        saggs = _sc_scatter(msg, w9, dst)
        aggs = jnp.concatenate(
            [saggs[:, 0:128], jaggs[:, 128:384]], axis=-1)
    else:
        aggs = _sc_scatter(msg, w9, dst)
    return _update(node_features, aggs, Wu1, bu1, Wu2, bu2, ln_gamma, ln_beta)


# speedup vs baseline: 1.1003x; 1.0075x over previous
import functools
import jax
import jax.numpy as jnp
from jax import lax
from jax.experimental import pallas as pl
from jax.experimental.pallas import tpu as pltpu

_N = 10000
_DEBUG_JNP_SCATTER = False
_E = 160000
_H = 128
_NB = 16


def _silu(x):
    return x * jax.nn.sigmoid(x)


def _proj_body(x_ref, ws_ref, bs_ref, wd_ref, bd_ref, ps_ref, pd_ref):
    x = x_ref[...]
    ps_ref[...] = jnp.dot(x, ws_ref[...], preferred_element_type=jnp.float32) + bs_ref[...]
    pd_ref[...] = jnp.dot(x, wd_ref[...], preferred_element_type=jnp.float32) + bd_ref[...]


def _proj(x, W_src, b_src, W_dst, b_dst):
    blk = 2000
    grid = _N // blk
    return pl.pallas_call(
        _proj_body,
        grid=(grid,),
        in_specs=[
            pl.BlockSpec((blk, _H), lambda i: (i, 0)),
            pl.BlockSpec((_H, _H), lambda i: (0, 0)),
            pl.BlockSpec((1, _H), lambda i: (0, 0)),
            pl.BlockSpec((_H, _H), lambda i: (0, 0)),
            pl.BlockSpec((1, _H), lambda i: (0, 0)),
        ],
        out_specs=[
            pl.BlockSpec((blk, _H), lambda i: (i, 0)),
            pl.BlockSpec((blk, _H), lambda i: (i, 0)),
        ],
        out_shape=[
            jax.ShapeDtypeStruct((_N, _H), jnp.float32),
            jax.ShapeDtypeStruct((_N, _H), jnp.float32),
        ],
    )(x, W_src.T, b_src[None, :], W_dst.T, b_dst[None, :])


def _edge_mlp_body(gs_ref, gd_ref, rbf_ref, b1_ref, b2_ref,
                   a1_ref, a2_ref, a3_ref, bf1_ref,
                   w2_ref, bf2_ref, msg_ref, w9_ref):
    t = jnp.dot(gs_ref[...], a1_ref[...], preferred_element_type=jnp.float32)
    t += jnp.dot(gd_ref[...], a2_ref[...], preferred_element_type=jnp.float32)
    t += jnp.dot(rbf_ref[...], a3_ref[...], preferred_element_type=jnp.float32)
    t += bf1_ref[...]
    t = _silu(t)
    msg_ref[...] = (
        jnp.dot(t, w2_ref[...], preferred_element_type=jnp.float32) + bf2_ref[...]
    )
    blk = b1_ref.shape[0]
    w9_ref[...] = jnp.concatenate(
        [jnp.full((blk, 1), 1.0 / 16.0, jnp.float32), b1_ref[...], b2_ref[...],
         jnp.zeros((blk, _H - 9), jnp.float32)], axis=-1)


def _edge_mlp(gs, gd, rbf, basis_1, basis_2, Wf1, bf1, Wf2, bf2):
    A1 = Wf1[:, 0:_H].T
    A2 = Wf1[:, _H:2 * _H].T
    A3 = Wf1[:, 2 * _H:].T
    blk = 2000
    grid = _E // blk
    return pl.pallas_call(
        _edge_mlp_body,
        grid=(grid,),
        in_specs=[
            pl.BlockSpec((blk, _H), lambda i: (i, 0)),
            pl.BlockSpec((blk, _H), lambda i: (i, 0)),
            pl.BlockSpec((blk, _NB), lambda i: (i, 0)),
            pl.BlockSpec((blk, 3), lambda i: (i, 0)),
            pl.BlockSpec((blk, 5), lambda i: (i, 0)),
            pl.BlockSpec((_H, _H), lambda i: (0, 0)),
            pl.BlockSpec((_H, _H), lambda i: (0, 0)),
            pl.BlockSpec((_NB, _H), lambda i: (0, 0)),
            pl.BlockSpec((1, _H), lambda i: (0, 0)),
            pl.BlockSpec((_H, _H), lambda i: (0, 0)),
            pl.BlockSpec((1, _H), lambda i: (0, 0)),
        ],
        out_specs=[
            pl.BlockSpec((blk, _H), lambda i: (i, 0)),
            pl.BlockSpec((blk, _H), lambda i: (i, 0)),
        ],
        out_shape=[
            jax.ShapeDtypeStruct((_E, _H), jnp.float32),
            jax.ShapeDtypeStruct((_E, _H), jnp.float32),
        ],
    )(gs, gd, rbf, basis_1, basis_2, A1, A2, A3, bf1[None, :], Wf2.T,
      bf2[None, :])


def _update_body(x_ref, aggs_ref, u1_ref, bu1_ref, u2_ref, bu2_ref,
                 g_ref, b_ref, out_ref):
    x = x_ref[...]
    s = aggs_ref[:, 0:_H]
    i1 = aggs_ref[:, _H:2 * _H]
    i2 = aggs_ref[:, 2 * _H:3 * _H]
    u1 = u1_ref[...]
    t = jnp.dot(x, u1[0:_H, :], preferred_element_type=jnp.float32)
    t += jnp.dot(s, u1[_H:2 * _H, :], preferred_element_type=jnp.float32)
    t += jnp.dot(i1, u1[2 * _H:3 * _H, :], preferred_element_type=jnp.float32)
    t += jnp.dot(i2, u1[3 * _H:4 * _H, :], preferred_element_type=jnp.float32)
    t += bu1_ref[...]
    t = _silu(t)
    upd = jnp.dot(t, u2_ref[...], preferred_element_type=jnp.float32) + bu2_ref[...]
    y = x + upd
    mu = jnp.mean(y, axis=-1, keepdims=True)
    var = jnp.mean((y - mu) ** 2, axis=-1, keepdims=True)
    out_ref[...] = (y - mu) * lax.rsqrt(var + 1e-5) * g_ref[...] + b_ref[...]


def _update(x, aggs, Wu1, bu1, Wu2, bu2, ln_gamma, ln_beta):
    blk = 2000
    grid = _N // blk
    return pl.pallas_call(
        _update_body,
        grid=(grid,),
        in_specs=[
            pl.BlockSpec((blk, _H), lambda i: (i, 0)),
            pl.BlockSpec((blk, 3 * _H), lambda i: (i, 0)),
            pl.BlockSpec((4 * _H, 2 * _H), lambda i: (0, 0)),
            pl.BlockSpec((1, 2 * _H), lambda i: (0, 0)),
            pl.BlockSpec((2 * _H, _H), lambda i: (0, 0)),
            pl.BlockSpec((1, _H), lambda i: (0, 0)),
            pl.BlockSpec((1, _H), lambda i: (0, 0)),
            pl.BlockSpec((1, _H), lambda i: (0, 0)),
        ],
        out_specs=pl.BlockSpec((blk, _H), lambda i: (i, 0)),
        out_shape=jax.ShapeDtypeStruct((_N, _H), jnp.float32),
    )(x, aggs, Wu1.T, bu1[None, :], Wu2.T, bu2[None, :],
      ln_gamma[None, :], ln_beta[None, :])


from jax.experimental.pallas import tpu_sc as plsc

_NC = 2
_NS = 16
_NW = _NC * _NS
_EPW = _E // _NW
_GB = 200

_ES = _E // _NS
_NCHUNK = 28
_CHUNK = 384
_AROWS = 392
_DUMP = 384
_RPT = 24
_K = 16
_HALVES = (2496, 2496, 2496, 2512)
_CAP = 2512 // _K + 3


def _sc_gather(P_src, P_dst, src, dst):
    mesh = plsc.VectorSubcoreMesh(core_axis_name="c", subcore_axis_name="s")

    @functools.partial(
        pl.kernel,
        compiler_params=pltpu.CompilerParams(needs_layout_passes=False),
        out_type=[
            jax.ShapeDtypeStruct((_E, _H), jnp.float32),
            jax.ShapeDtypeStruct((_E, _H), jnp.float32),
        ],
        mesh=mesh,
        scratch_types=[
            pltpu.VMEM((_GB,), jnp.int32),
            pltpu.VMEM((_GB,), jnp.int32),
            pltpu.VMEM((_GB, _H), jnp.float32),
            pltpu.VMEM((_GB, _H), jnp.float32),
            pltpu.SemaphoreType.DMA,
            pltpu.SemaphoreType.DMA,
        ],
    )
    def gather_k(ps_hbm, pd_hbm, src_hbm, dst_hbm, gs_hbm, gd_hbm,
                 si_v, di_v, rs_v, rd_v, sem1, sem2):
        wid = lax.axis_index("s") * _NC + lax.axis_index("c")
        base = wid * _EPW

        def body(b, carry):
            off = base + b * _GB
            pltpu.sync_copy(src_hbm.at[pl.ds(off, _GB)], si_v)
            pltpu.sync_copy(dst_hbm.at[pl.ds(off, _GB)], di_v)
            c1 = pltpu.async_copy(ps_hbm.at[si_v], rs_v, sem1)
            c2 = pltpu.async_copy(pd_hbm.at[di_v], rd_v, sem2)
            c1.wait()
            c2.wait()
            pltpu.sync_copy(rs_v, gs_hbm.at[pl.ds(off, _GB)])
            pltpu.sync_copy(rd_v, gd_hbm.at[pl.ds(off, _GB)])
            return carry

        lax.fori_loop(0, _EPW // _GB, body, 0)

    return gather_k(P_src, P_dst, src, dst)


def _sc_scatter(msg, w9, dst):
    mesh = plsc.VectorSubcoreMesh(core_axis_name="c", subcore_axis_name="s")

    @functools.partial(
        pl.kernel,
        compiler_params=pltpu.CompilerParams(needs_layout_passes=False),
        out_type=jax.ShapeDtypeStruct((_N, 3 * _H), jnp.float32),
        mesh=mesh,
        scratch_types=[
            pltpu.VMEM((2512,), jnp.int32),
            pltpu.VMEM((_CAP, _K), jnp.int32),
            pltpu.VMEM((_CAP, _K), jnp.int32),
            pltpu.VMEM((_K, _H), jnp.float32),
            pltpu.VMEM((_K, _H), jnp.float32),
            pltpu.VMEM((_K, _H), jnp.float32),
            pltpu.VMEM((_K, _H), jnp.float32),
            pltpu.VMEM((9, _K, _H), jnp.float32),
            pltpu.VMEM((9, _K, _H), jnp.float32),
            pltpu.VMEM((8, 3 * _H), jnp.float32),
        ] + [pltpu.VMEM_SHARED((_AROWS, _H), jnp.float32)] * 9 + [
            pltpu.SemaphoreType.DMA,
            pltpu.SemaphoreType.DMA,
            pltpu.SemaphoreType.DMA,
            pltpu.SemaphoreType.DMA,
        ],
    )
    def scatter_k(msg_hbm, w9_hbm, dst_hbm, out_hbm,
                  dst_v, eid_v, dl_v, msg_v, w_v, msg2_v, w2_v, srow_v,
                  srow2_v, out_v,
                  a0, a1, a2, a3, a4, a5, a6, a7, a8, sem1, sem2, sem3,
                  sem4):
        accs = [a0, a1, a2, a3, a4, a5, a6, a7, a8]
        cid = lax.axis_index("c")
        sid = lax.axis_index("s")
        sbase = sid * _ES
        iota16 = lax.broadcasted_iota(jnp.int32, (16,), 0)

        def chunk_body(j, carry):
            kk = cid * (_NCHUNK // _NC) + j
            lo = kk * _CHUNK
            hi = jnp.minimum(_N, lo + _CHUNK)
            sz = hi - lo

            for r in range(_K):
                for q in range(_H // 16):
                    srow_v[0, r, pl.ds(q * 16, 16)] = jnp.zeros(
                        (16,), jnp.float32)
            zbase = sid * _RPT
            for c in range(9):
                for i in range(_RPT // 8):
                    pltpu.sync_copy(srow_v.at[0, pl.ds(0, 8)],
                                    accs[c].at[pl.ds(zbase + 8 * i, 8)])
            plsc.subcore_barrier()

            hoff = 0
            for hlen in _HALVES:
                hbase = sbase + hoff
                pltpu.sync_copy(dst_hbm.at[pl.ds(hbase, hlen)],
                                dst_v.at[pl.ds(0, hlen)])

                def compact_group(off, cnt, d16):
                    m = (d16 >= lo) & (d16 < hi)
                    mi = jnp.where(m, 1, 0)
                    raw = cnt + plsc.cumsum(mi) - 1
                    idx = jnp.where(m, raw, (_CAP - 1) * 16 + iota16)
                    eids = iota16 + (hbase + off)
                    plsc.store_scatter(eid_v, [idx >> 4, idx & 15], eids,
                                       mask=m)
                    plsc.store_scatter(dl_v, [idx >> 4, idx & 15],
                                       jnp.where(m, d16 - lo, _DUMP), mask=m)
                    return jnp.sum(mi)

                def compact_pair(i, cnt):
                    d16a = dst_v[pl.ds(i * 32, 16)]
                    d16b = dst_v[pl.ds(i * 32 + 16, 16)]
                    ma = (d16a >= lo) & (d16a < hi)
                    sa = jnp.sum(jnp.where(ma, 1, 0))
                    na = compact_group(i * 32, cnt, d16a)
                    nb_ = compact_group(i * 32 + 16, cnt + sa, d16b)
                    return cnt + na + nb_

                cnt = lax.fori_loop(0, hlen // 32, compact_pair, 0)
                if hlen % 32:
                    cnt = cnt + compact_group(
                        hlen - 16, cnt, dst_v[pl.ds(hlen - 16, 16)])
                idxp = cnt + iota16
                plsc.store_scatter(eid_v, [idxp >> 4, idxp & 15],
                                   jnp.zeros((16,), jnp.int32))
                plsc.store_scatter(dl_v, [idxp >> 4, idxp & 15],
                                   jnp.full((16,), _DUMP, jnp.int32))

                def batch_body(b, carry):
                    c1 = pltpu.async_copy(msg_hbm.at[eid_v.at[b]], msg_v,
                                          sem1)
                    c2 = pltpu.async_copy(w9_hbm.at[eid_v.at[b]], w_v, sem2)
                    c1.wait()
                    c2.wait()

                    def row_body(r, carry2):
                        wvec = w_v[r, pl.ds(0, 16)]
                        for c in range(9):
                            wc = wvec[c]
                            for q in range(_H // 16):
                                srow_v[c, r, pl.ds(q * 16, 16)] = (
                                    wc * msg_v[r, pl.ds(q * 16, 16)])
                        return carry2

                    lax.fori_loop(0, _K, row_body, 0)
                    descs = []
                    for c in range(9):
                        d = pltpu.make_async_copy(
                            srow_v.at[c], accs[c].at[dl_v.at[b]], sem3)
                        d.start(add=True)
                        descs.append(d)
                    for d in descs:
                        d.wait()
                    return carry

                nb = (cnt + _K - 1) // _K
                lax.fori_loop(0, nb, batch_body, 0)
                hoff += hlen
            plsc.subcore_barrier()

            rstart = sid * _RPT
            ng = jnp.clip((sz - rstart) // 8, 0, _RPT // 8)

            def red_body(g, carry):
                row = rstart + 8 * g
                descs = [
                    pltpu.async_copy(accs[c].at[pl.ds(row, 8)],
                                     srow_v.at[c, pl.ds(0, 8)], sem3)
                    for c in range(9)
                ]
                for d in descs:
                    d.wait()

                def rrow(r, carry2):
                    for q in range(_H // 16):
                        s0 = srow_v[0, r, pl.ds(q * 16, 16)]
                        t1 = srow_v[1, r, pl.ds(q * 16, 16)]
                        t2 = srow_v[2, r, pl.ds(q * 16, 16)]
                        t3 = srow_v[3, r, pl.ds(q * 16, 16)]
                        i1 = (t1 * t1 + t2 * t2 + t3 * t3) * (1.0 / 3.0)
                        u1 = srow_v[4, r, pl.ds(q * 16, 16)]
                        u2 = srow_v[5, r, pl.ds(q * 16, 16)]
                        u3 = srow_v[6, r, pl.ds(q * 16, 16)]
                        u4 = srow_v[7, r, pl.ds(q * 16, 16)]
                        u5 = srow_v[8, r, pl.ds(q * 16, 16)]
                        i2 = (u1 * u1 + u2 * u2 + u3 * u3 + u4 * u4
                              + u5 * u5) * (1.0 / 5.0)
                        out_v[r, pl.ds(q * 16, 16)] = s0
                        out_v[r, pl.ds(_H + q * 16, 16)] = i1
                        out_v[r, pl.ds(2 * _H + q * 16, 16)] = i2
                    return carry2

                lax.fori_loop(0, 8, rrow, 0)
                pltpu.sync_copy(out_v, out_hbm.at[pl.ds(lo + row, 8)])
                return carry

            lax.fori_loop(0, ng, red_body, 0)
            plsc.subcore_barrier()
            return carry

        lax.fori_loop(0, _NCHUNK // _NC, chunk_body, 0)

    return scatter_k(msg, w9, dst)


def kernel(node_features, edge_index, edge_rbf, basis_0, basis_1, basis_2,
           W_src, b_src, W_dst, b_dst, Wf1, bf1, Wf2, bf2,
           Wu1, bu1, Wu2, bu2, ln_gamma, ln_beta):
    src = edge_index[0]
    dst = edge_index[1]
    P_src, P_dst = _proj(node_features, W_src, b_src, W_dst, b_dst)
    gs, gd = _sc_gather(P_src, P_dst, src, dst)
    msg, w9 = _edge_mlp(gs, gd, edge_rbf, basis_1, basis_2, Wf1, bf1, Wf2, bf2)
    if _DEBUG_JNP_SCATTER:
        prod = msg[:, None, :] * w9[:, 0:9, None]
        sums = jax.ops.segment_sum(prod, dst, num_segments=_N)
        jaggs = jnp.concatenate(
            [sums[:, 0, :], jnp.mean(sums[:, 1:4, :] ** 2, axis=1),
             jnp.mean(sums[:, 4:9, :] ** 2, axis=1)], axis=-1)
        saggs = _sc_scatter(msg, w9, dst)
        aggs = jnp.concatenate(
            [saggs[:, 0:128], jaggs[:, 128:384]], axis=-1)
    else:
        aggs = _sc_scatter(msg, w9, dst)
    return _update(node_features, aggs, Wu1, bu1, Wu2, bu2, ln_gamma, ln_beta)
